# baseline (speedup 1.0000x reference)
import math

import jax
import jax.numpy as jnp
from jax.experimental import pallas as pl

DIM = 2048
RANK = 32
NUM_ENTRIES = 64
_SCALE = 2.0 / math.sqrt(RANK)

_E_BLK = 8
_P = 256


def _adalora_block(idx_ref, st_ref, d_ref, u_ref, o_ref):
    j = pl.program_id(0)
    e0 = j * _E_BLK
    yt = jnp.dot(d_ref[...], st_ref[...], preferred_element_type=jnp.float32)
    eid = jax.lax.broadcasted_iota(jnp.int32, (_E_BLK * RANK, _P), 0) // RANK + e0
    keep = eid == idx_ref[...]
    yt = jnp.where(keep, yt, 0.0) * _SCALE
    z = jax.lax.dot_general(yt, u_ref[...], (((0,), (0,)), ((), ())),
                            preferred_element_type=jnp.float32)

    @pl.when(j == 0)
    def _init():
        o_ref[...] = z

    @pl.when(j > 0)
    def _acc():
        o_ref[...] += z


def kernel(slots, indices, down_proj_values, up_proj_values):
    b, k, d = slots.shape
    p = b * k
    st = jnp.transpose(slots.reshape(p, d))
    idx = indices.reshape(1, p).astype(jnp.int32)
    dt = jnp.transpose(down_proj_values, (0, 2, 1)).reshape(NUM_ENTRIES * RANK, d)
    u2 = up_proj_values.reshape(NUM_ENTRIES * RANK, d)

    out = pl.pallas_call(
        _adalora_block,
        grid=(NUM_ENTRIES // _E_BLK,),
        in_specs=[
            pl.BlockSpec((1, p), lambda j: (0, 0)),
            pl.BlockSpec((d, p), lambda j: (0, 0)),
            pl.BlockSpec((_E_BLK * RANK, d), lambda j: (j, 0)),
            pl.BlockSpec((_E_BLK * RANK, d), lambda j: (j, 0)),
        ],
        out_specs=pl.BlockSpec((p, d), lambda j: (0, 0)),
        out_shape=jax.ShapeDtypeStruct((p, d), jnp.float32),
    )(idx, st, dt, u2)
    return out.reshape(b, k, d)


# speedup vs baseline: 1.2000x; 1.2000x over previous
import math

import jax
import jax.numpy as jnp
from jax.experimental import pallas as pl
from jax.experimental.pallas import tpu as pltpu

DIM = 2048
RANK = 32
NUM_ENTRIES = 64
_SCALE = 2.0 / math.sqrt(RANK)

_E_BLK = 16
_P = 256


def _adalora_block(idx_ref, s_ref, d_ref, u_ref, o_ref, st_scr):
    j = pl.program_id(0)
    e0 = j * _E_BLK

    @pl.when(j == 0)
    def _stage_st():
        st_scr[...] = jnp.transpose(s_ref[...] * _SCALE)

    yt = jnp.dot(d_ref[...], st_scr[...], preferred_element_type=jnp.float32)
    eid = jax.lax.broadcasted_iota(jnp.int32, (_E_BLK * RANK, _P), 0) // RANK + e0
    keep = eid == idx_ref[...]
    yt = jnp.where(keep, yt, 0.0)
    z = jax.lax.dot_general(yt, u_ref[...], (((0,), (0,)), ((), ())),
                            preferred_element_type=jnp.float32)

    @pl.when(j == 0)
    def _init():
        o_ref[...] = z

    @pl.when(j > 0)
    def _acc():
        o_ref[...] += z


def kernel(slots, indices, down_proj_values, up_proj_values):
    b, k, d = slots.shape
    p = b * k
    s2 = slots.reshape(p, d)
    idx = indices.reshape(1, p).astype(jnp.int32)
    dt = jnp.transpose(down_proj_values, (0, 2, 1)).reshape(NUM_ENTRIES * RANK, d)
    u2 = up_proj_values.reshape(NUM_ENTRIES * RANK, d)

    out = pl.pallas_call(
        _adalora_block,
        grid=(NUM_ENTRIES // _E_BLK,),
        in_specs=[
            pl.BlockSpec((1, p), lambda j: (0, 0)),
            pl.BlockSpec((p, d), lambda j: (0, 0)),
            pl.BlockSpec((_E_BLK * RANK, d), lambda j: (j, 0)),
            pl.BlockSpec((_E_BLK * RANK, d), lambda j: (j, 0)),
        ],
        out_specs=pl.BlockSpec((p, d), lambda j: (0, 0)),
        out_shape=jax.ShapeDtypeStruct((p, d), jnp.float32),
        scratch_shapes=[pltpu.VMEM((d, p), jnp.float32)],
    )(idx, s2, dt, u2)
    return out.reshape(b, k, d)
